# Initial kernel scaffold; baseline (speedup 1.0000x reference)
import jax
import jax.numpy as jnp
from jax.experimental import pallas as pl


def kernel(x, freq_pos, time_pos, freq_embedding, time_embedding, W, bias):
    raise NotImplementedError("write your pallas kernel here")


# speedup vs baseline: 1.8450x; 1.8450x over previous
import functools

import jax
import jax.numpy as jnp
from jax import lax
from jax.experimental import pallas as pl
from jax.experimental.pallas import tpu as pltpu
from jax.experimental.pallas import tpu_sc as plsc

B, N, D = 32, 512, 768
NUM_FREQ, NUM_TIME = 8, 64
ROWS = B * N
NC, NS = 2, 16
NW = NC * NS
RPT = ROWS // NW
CH = 64
NCH = RPT // CH


def _table_idx_body(fe_ref, te_ref, wa_ref, wb_ref, bias_ref, fp_ref, tp_ref,
                    table_ref, idx_ref):
    dn = (((1,), (1,)), ((), ()))
    wf = lax.dot_general(fe_ref[...], wa_ref[...], dn,
                         preferred_element_type=jnp.float32)
    wt = lax.dot_general(te_ref[...], wb_ref[...], dn,
                         preferred_element_type=jnp.float32)
    base = wt + bias_ref[...]
    for f in range(NUM_FREQ):
        table_ref[f * NUM_TIME:(f + 1) * NUM_TIME, :] = base + wf[f:f + 1, :]
    idx_ref[...] = fp_ref[...] * NUM_TIME + tp_ref[...]


@jax.jit
def _build_table_idx(fe, te, wa, wb, bias2d, fp, tp):
    return pl.pallas_call(
        _table_idx_body,
        out_shape=[
            jax.ShapeDtypeStruct((NUM_FREQ * NUM_TIME, D), jnp.float32),
            jax.ShapeDtypeStruct((B, N), jnp.int32),
        ],
    )(fe, te, wa, wb, bias2d, fp, tp)


def _sc_body(x_hbm, idx_hbm, table_hbm, out_hbm, idx_v, xbuf, gbuf, gsem):
    wid = lax.axis_index("s") * NC + lax.axis_index("c")
    base = wid * RPT
    pltpu.sync_copy(idx_hbm.at[pl.ds(base, RPT)], idx_v)
    for c in range(NCH):
        rb = base + c * CH
        gather = pltpu.async_copy(
            table_hbm.at[idx_v.at[pl.ds(c * CH, CH)]], gbuf, gsem)
        pltpu.sync_copy(x_hbm.at[pl.ds(rb, CH)], xbuf)
        gather.wait()

        def row(r, carry):
            for j in range(D // 16):
                sl = pl.ds(j * 16, 16)
                xbuf[r, sl] = xbuf[r, sl] + gbuf[r, sl]
            return carry

        lax.fori_loop(0, CH, row, 0)
        pltpu.sync_copy(xbuf, out_hbm.at[pl.ds(rb, CH)])


@jax.jit
def _sc_gather_add(xf, idx_flat, table):
    run = pl.kernel(
        _sc_body,
        out_type=jax.ShapeDtypeStruct((ROWS, D), jnp.float32),
        mesh=plsc.VectorSubcoreMesh(core_axis_name="c", subcore_axis_name="s"),
        scratch_types=[
            pltpu.VMEM((RPT,), jnp.int32),
            pltpu.VMEM((CH, D), jnp.float32),
            pltpu.VMEM((CH, D), jnp.float32),
            pltpu.SemaphoreType.DMA,
        ],
    )
    return run(xf, idx_flat, table)


def kernel(x, freq_pos, time_pos, freq_embedding, time_embedding, W, bias):
    wa = W[:, :D]
    wb = W[:, D:]
    table, idx = _build_table_idx(freq_embedding, time_embedding, wa, wb,
                                  bias.reshape(1, D),
                                  freq_pos.astype(jnp.int32),
                                  time_pos.astype(jnp.int32))
    out = _sc_gather_add(x.reshape(ROWS, D), idx.reshape(ROWS), table)
    return out.reshape(B, N, D)


# speedup vs baseline: 2.3620x; 1.2802x over previous
import functools

import jax
import jax.numpy as jnp
from jax import lax
from jax.experimental import pallas as pl
from jax.experimental.pallas import tpu as pltpu
from jax.experimental.pallas import tpu_sc as plsc

B, N, D = 32, 512, 768
NUM_FREQ, NUM_TIME = 8, 64
ROWS = B * N
NC, NS = 2, 16
NW = NC * NS
RPT = ROWS // NW
CH = 32
NCH = RPT // CH


def _table_idx_body(fe_ref, te_ref, wa_ref, wb_ref, bias_ref, fp_ref, tp_ref,
                    table_ref, idx_ref):
    dn = (((1,), (1,)), ((), ()))
    wf = lax.dot_general(fe_ref[...], wa_ref[...], dn,
                         preferred_element_type=jnp.float32)
    wt = lax.dot_general(te_ref[...], wb_ref[...], dn,
                         preferred_element_type=jnp.float32)
    base = wt + bias_ref[...]
    for f in range(NUM_FREQ):
        table_ref[f * NUM_TIME:(f + 1) * NUM_TIME, :] = base + wf[f:f + 1, :]
    idx_ref[...] = fp_ref[...] * NUM_TIME + tp_ref[...]


@jax.jit
def _build_table_idx(fe, te, wa, wb, bias2d, fp, tp):
    return pl.pallas_call(
        _table_idx_body,
        out_shape=[
            jax.ShapeDtypeStruct((NUM_FREQ * NUM_TIME, D), jnp.float32),
            jax.ShapeDtypeStruct((B, N), jnp.int32),
        ],
    )(fe, te, wa, wb, bias2d, fp, tp)


def _sc_body(x_hbm, idx_hbm, table_hbm, out_hbm,
             idx_v, xb0, xb1, gb0, gb1,
             xs0, xs1, gs0, gs1, os0, os1):
    cid = lax.axis_index("c")
    sid = lax.axis_index("s")
    wid = sid * NC + cid
    base = wid * RPT

    pltpu.sync_copy(idx_hbm.at[pl.ds(base, RPT)], idx_v)

    xb, gb = [xb0, xb1], [gb0, gb1]
    xs, gs, osm = [xs0, xs1], [gs0, gs1], [os0, os1]
    xcp, gcp, ocp = [None, None], [None, None], [None, None]

    def issue(c):
        s = c % 2
        xcp[s] = pltpu.async_copy(
            x_hbm.at[pl.ds(base + c * CH, CH)], xb[s], xs[s])
        gcp[s] = pltpu.async_copy(
            table_hbm.at[idx_v.at[pl.ds(c * CH, CH)]], gb[s], gs[s])

    issue(0)
    for c in range(NCH):
        s = c % 2
        if c + 1 < NCH:
            if ocp[1 - s] is not None:
                ocp[1 - s].wait()
            issue(c + 1)
        xcp[s].wait()
        gcp[s].wait()

        def row(r, carry):
            for j in range(D // 16):
                sl = pl.ds(j * 16, 16)
                xb[s][r, sl] = xb[s][r, sl] + gb[s][r, sl]
            return carry

        lax.fori_loop(0, CH, row, 0)
        ocp[s] = pltpu.async_copy(
            xb[s], out_hbm.at[pl.ds(base + c * CH, CH)], osm[s])
    ocp[0].wait()
    ocp[1].wait()


@jax.jit
def _sc_gather_add(xf, idx_flat, table):
    run = pl.kernel(
        _sc_body,
        out_type=jax.ShapeDtypeStruct((ROWS, D), jnp.float32),
        mesh=plsc.VectorSubcoreMesh(core_axis_name="c", subcore_axis_name="s"),
        scratch_types=[
            pltpu.VMEM((RPT,), jnp.int32),
            pltpu.VMEM((CH, D), jnp.float32),
            pltpu.VMEM((CH, D), jnp.float32),
            pltpu.VMEM((CH, D), jnp.float32),
            pltpu.VMEM((CH, D), jnp.float32),
            pltpu.SemaphoreType.DMA,
            pltpu.SemaphoreType.DMA,
            pltpu.SemaphoreType.DMA,
            pltpu.SemaphoreType.DMA,
            pltpu.SemaphoreType.DMA,
            pltpu.SemaphoreType.DMA,
        ],
    )
    return run(xf, idx_flat, table)


def kernel(x, freq_pos, time_pos, freq_embedding, time_embedding, W, bias):
    wa = W[:, :D]
    wb = W[:, D:]
    table, idx = _build_table_idx(freq_embedding, time_embedding, wa, wb,
                                  bias.reshape(1, D),
                                  freq_pos.astype(jnp.int32),
                                  time_pos.astype(jnp.int32))
    out = _sc_gather_add(x.reshape(ROWS, D), idx.reshape(ROWS), table)
    return out.reshape(B, N, D)
